# Initial kernel scaffold; baseline (speedup 1.0000x reference)
import functools

import jax
import jax.numpy as jnp
from jax import lax
from jax.experimental import pallas as pl
from jax.experimental.pallas import tpu as pltpu
from jax.experimental.pallas import tpu_sc as plsc

N = 10000
E = 320000
D_IN = 128
DH = 64
NCLS = 7

NC = 2
NS = 16
NW = NC * NS
CB = 128
CW = 80
EW = CW * CB
EPAD = NW * EW
NP = 10240
TR = NP // NS
BLK = 1024
GRID = NP // BLK

_mesh = plsc.VectorSubcoreMesh(core_axis_name="c", subcore_axis_name="s")


@functools.partial(
    pl.kernel,
    out_type=jax.ShapeDtypeStruct((NC * NP,), jnp.float32),
    mesh=_mesh,
    scratch_types=[
        pltpu.VMEM((CW, CB), jnp.int32),
        pltpu.VMEM((TR,), jnp.float32),
        pltpu.VMEM((CB,), jnp.float32),
        pltpu.VMEM_SHARED((NP,), jnp.float32),
    ],
)
def _sc_degree(dst_hbm, out_hbm, didx, zbuf, ones, deg_sh):
    c = lax.axis_index("c")
    s = lax.axis_index("s")
    wid = c * NS + s
    base = s * TR

    def fill_z(i, _):
        zbuf[pl.ds(i * 16, 16)] = jnp.zeros((16,), jnp.float32)
        return 0

    lax.fori_loop(0, TR // 16, fill_z, 0)

    def fill_o(i, _):
        ones[pl.ds(i * 16, 16)] = jnp.ones((16,), jnp.float32)
        return 0

    lax.fori_loop(0, CB // 16, fill_o, 0)

    pltpu.sync_copy(dst_hbm.at[wid], didx)
    pltpu.sync_copy(zbuf, deg_sh.at[pl.ds(base, TR)])
    plsc.subcore_barrier()

    def body(j, _):
        pltpu.sync_copy(ones, deg_sh.at[didx.at[j]], add=True)
        return 0

    lax.fori_loop(0, CW, body, 0)
    plsc.subcore_barrier()
    pltpu.sync_copy(deg_sh.at[pl.ds(base, TR)],
                    out_hbm.at[pl.ds(c * NP + base, TR)])


@functools.partial(
    pl.kernel,
    out_type=jax.ShapeDtypeStruct((NC * NP, DH), jnp.float32),
    mesh=_mesh,
    scratch_types=[
        pltpu.VMEM((CW, CB), jnp.int32),
        pltpu.VMEM((CW, CB), jnp.int32),
        pltpu.VMEM((CB, DH), jnp.float32),
        pltpu.VMEM_SHARED((NP, DH), jnp.float32),
        pltpu.VMEM_SHARED((NP, DH), jnp.float32),
        pltpu.SemaphoreType.DMA,
    ],
)
def _sc_aggregate(hs_hbm, src_hbm, dst_hbm, out_hbm,
                  sidx, didx, rows, hs_sh, acc_sh, sem):
    c = lax.axis_index("c")
    s = lax.axis_index("s")
    wid = c * NS + s
    base = s * TR

    def fill_z(i, _):
        rows[i // 4, pl.ds((i % 4) * 16, 16)] = jnp.zeros((16,), jnp.float32)
        return 0

    lax.fori_loop(0, CB * DH // 16, fill_z, 0)
    for t in range(TR // CB):
        pltpu.sync_copy(rows, acc_sh.at[pl.ds(base + t * CB, CB)])

    pltpu.sync_copy(src_hbm.at[wid], sidx)
    pltpu.sync_copy(dst_hbm.at[wid], didx)
    pltpu.sync_copy(hs_hbm.at[pl.ds(base, TR)], hs_sh.at[pl.ds(base, TR)])
    plsc.subcore_barrier()

    def body(j, _):
        pltpu.async_copy(hs_sh.at[sidx.at[j]], rows, sem).wait()
        pltpu.sync_copy(rows, acc_sh.at[didx.at[j]], add=True)
        return 0

    lax.fori_loop(0, CW, body, 0)
    plsc.subcore_barrier()
    pltpu.sync_copy(acc_sh.at[pl.ds(base, TR)],
                    out_hbm.at[pl.ds(c * NP + base, TR)])


def _tc1_body(x_ref, degc_ref, w1_ref, hs_ref):
    dis = lax.rsqrt(degc_ref[:, 0:1] + degc_ref[:, 1:2] + 1.0)
    h = jnp.dot(x_ref[...], w1_ref[...],
                preferred_element_type=jnp.float32,
                precision=lax.Precision.HIGHEST)
    hs_ref[...] = h * dis


def _tc2_body(pa_ref, pb_ref, hs_ref, degc_ref, b1_ref, w2_ref, hs2_ref):
    dis = lax.rsqrt(degc_ref[:, 0:1] + degc_ref[:, 1:2] + 1.0)
    agg = (pa_ref[...] + pb_ref[...] + hs_ref[...]) * dis
    y = jnp.maximum(agg + b1_ref[...], 0.0)
    hs2_ref[...] = jnp.dot(y, w2_ref[...],
                           preferred_element_type=jnp.float32,
                           precision=lax.Precision.HIGHEST) * dis


def _tc3_body(pa_ref, pb_ref, hs_ref, degc_ref, b2_ref,
              wf1_ref, bf1_ref, wf2_ref, bf2_ref, wf3_ref, bf3_ref, out_ref):
    dis = lax.rsqrt(degc_ref[:, 0:1] + degc_ref[:, 1:2] + 1.0)
    agg = (pa_ref[...] + pb_ref[...] + hs_ref[...]) * dis
    y = jnp.maximum(agg + b2_ref[...], 0.0)
    z = jnp.maximum(jnp.dot(y, wf1_ref[...],
                            preferred_element_type=jnp.float32,
                            precision=lax.Precision.HIGHEST) + bf1_ref[...], 0.0)
    z = jnp.maximum(jnp.dot(z, wf2_ref[...],
                            preferred_element_type=jnp.float32,
                            precision=lax.Precision.HIGHEST) + bf2_ref[...], 0.0)
    logits = jnp.dot(z, wf3_ref[...],
                     preferred_element_type=jnp.float32,
                     precision=lax.Precision.HIGHEST) + bf3_ref[...]
    mask = lax.broadcasted_iota(jnp.int32, (BLK, 128), 1) < NCLS
    neg = jnp.float32(-1e30)
    m = jnp.max(jnp.where(mask, logits, neg), axis=1, keepdims=True)
    ex = jnp.where(mask, jnp.exp(logits - m), 0.0)
    lse = jnp.log(jnp.sum(ex, axis=1, keepdims=True))
    out_ref[...] = logits - m - lse


def _row_spec(d):
    return pl.BlockSpec((BLK, d), lambda i: (i, 0))


def _full_spec(r, d):
    return pl.BlockSpec((r, d), lambda i: (0, 0))


def kernel(x, edge_index, W1, b1, W2, b2, Wf1, bf1, Wf2, bf2, Wf3, bf3):
    src = edge_index[0]
    dst = edge_index[1]
    pad = N + (jnp.arange(EPAD - E, dtype=jnp.int32) % (NP - N))
    src_p = jnp.concatenate([src, pad]).reshape(NW, CW, CB)
    dst_p = jnp.concatenate([dst, pad]).reshape(NW, CW, CB)
    x_p = jnp.pad(x, ((0, NP - N), (0, 0)))

    deg_flat = _sc_degree(dst_p)
    degc = deg_flat.reshape(NC, NP).T

    hs1 = pl.pallas_call(
        _tc1_body,
        grid=(GRID,),
        in_specs=[_row_spec(D_IN), _row_spec(NC), _full_spec(D_IN, DH)],
        out_specs=_row_spec(DH),
        out_shape=jax.ShapeDtypeStruct((NP, DH), jnp.float32),
    )(x_p, degc, W1)

    part1 = _sc_aggregate(hs1, src_p, dst_p)
    pa1, pb1 = part1[:NP], part1[NP:]

    b1r = b1.reshape(1, DH)
    hs2 = pl.pallas_call(
        _tc2_body,
        grid=(GRID,),
        in_specs=[_row_spec(DH), _row_spec(DH), _row_spec(DH), _row_spec(NC),
                  _full_spec(1, DH), _full_spec(DH, DH)],
        out_specs=_row_spec(DH),
        out_shape=jax.ShapeDtypeStruct((NP, DH), jnp.float32),
    )(pa1, pb1, hs1, degc, b1r, W2)

    part2 = _sc_aggregate(hs2, src_p, dst_p)
    pa2, pb2 = part2[:NP], part2[NP:]

    wf3p = jnp.pad(Wf3, ((0, 0), (0, 128 - NCLS)))
    bf3p = jnp.pad(bf3, (0, 128 - NCLS)).reshape(1, 128)
    out = pl.pallas_call(
        _tc3_body,
        grid=(GRID,),
        in_specs=[_row_spec(DH), _row_spec(DH), _row_spec(DH), _row_spec(NC),
                  _full_spec(1, DH), _full_spec(DH, DH), _full_spec(1, DH),
                  _full_spec(DH, DH), _full_spec(1, DH),
                  _full_spec(DH, 128), _full_spec(1, 128)],
        out_specs=_row_spec(128),
        out_shape=jax.ShapeDtypeStruct((NP, 128), jnp.float32),
    )(pa2, pb2, hs2, degc, b2.reshape(1, DH), Wf1, bf1.reshape(1, DH),
      Wf2, bf2.reshape(1, DH), wf3p, bf3p)

    return out[:N, :NCLS]


# speedup vs baseline: 27.1778x; 27.1778x over previous
import functools

import jax
import jax.numpy as jnp
from jax import lax
from jax.experimental import pallas as pl
from jax.experimental.pallas import tpu as pltpu
from jax.experimental.pallas import tpu_sc as plsc

N = 10000
E = 320000
D_IN = 128
DH = 64
NCLS = 7

NC = 2
NS = 16
NW = NC * NS
CB = 128
CW = 80
EW = CW * CB
EPAD = NW * EW
NP = 10240
TR = NP // NS
BLK = 1024
GRID = NP // BLK

@functools.cache
def _get_sc_degree():
    mesh = plsc.VectorSubcoreMesh(core_axis_name="c", subcore_axis_name="s",
                                  num_cores=NC, num_subcores=NS)
    return pl.kernel(
        _sc_degree_body,
        out_type=jax.ShapeDtypeStruct((NC * NP,), jnp.float32),
        mesh=mesh,
        compiler_params=pltpu.CompilerParams(use_tc_tiling_on_sc=False),
        scratch_types=[
            pltpu.VMEM((CW, CB), jnp.int32),
            pltpu.VMEM((TR,), jnp.float32),
            pltpu.VMEM((CB,), jnp.float32),
            pltpu.VMEM_SHARED((NP,), jnp.float32),
        ],
    )


def _sc_degree_body(dst_hbm, out_hbm, didx, zbuf, ones, deg_sh):
    c = lax.axis_index("c")
    s = lax.axis_index("s")
    wid = c * NS + s
    base = s * TR

    def fill_z(i, _):
        zbuf[pl.ds(i * 16, 16)] = jnp.zeros((16,), jnp.float32)
        return 0

    lax.fori_loop(0, TR // 16, fill_z, 0)

    def fill_o(i, _):
        ones[pl.ds(i * 16, 16)] = jnp.ones((16,), jnp.float32)
        return 0

    lax.fori_loop(0, CB // 16, fill_o, 0)

    pltpu.sync_copy(dst_hbm.at[wid], didx)
    pltpu.sync_copy(zbuf, deg_sh.at[pl.ds(base, TR)])
    plsc.subcore_barrier()

    def body(j, _):
        pltpu.sync_copy(ones, deg_sh.at[didx.at[j]], add=True)
        return 0

    lax.fori_loop(0, CW, body, 0)
    plsc.subcore_barrier()
    pltpu.sync_copy(deg_sh.at[pl.ds(base, TR)],
                    out_hbm.at[pl.ds(c * NP + base, TR)])


@functools.cache
def _get_sc_aggregate():
    mesh = plsc.VectorSubcoreMesh(core_axis_name="c", subcore_axis_name="s",
                                  num_cores=NC, num_subcores=NS)
    return pl.kernel(
        _sc_aggregate_body,
        out_type=jax.ShapeDtypeStruct((NC * NP, DH), jnp.float32),
        mesh=mesh,
        compiler_params=pltpu.CompilerParams(use_tc_tiling_on_sc=False),
        scratch_types=[
            pltpu.VMEM((CW, CB), jnp.int32),
            pltpu.VMEM((CW, CB), jnp.int32),
            pltpu.VMEM((CB, DH), jnp.float32),
            pltpu.VMEM_SHARED((NP, DH), jnp.float32),
            pltpu.VMEM_SHARED((NP, DH), jnp.float32),
            pltpu.SemaphoreType.DMA,
        ],
    )


def _sc_aggregate_body(hs_hbm, src_hbm, dst_hbm, out_hbm,
                       sidx, didx, rows, hs_sh, acc_sh, sem):
    c = lax.axis_index("c")
    s = lax.axis_index("s")
    wid = c * NS + s
    base = s * TR

    def fill_z(i, _):
        rows[i // 4, pl.ds((i % 4) * 16, 16)] = jnp.zeros((16,), jnp.float32)
        return 0

    lax.fori_loop(0, CB * DH // 16, fill_z, 0)
    for t in range(TR // CB):
        pltpu.sync_copy(rows, acc_sh.at[pl.ds(base + t * CB, CB)])

    pltpu.sync_copy(src_hbm.at[wid], sidx)
    pltpu.sync_copy(dst_hbm.at[wid], didx)
    pltpu.sync_copy(hs_hbm.at[pl.ds(base, TR)], hs_sh.at[pl.ds(base, TR)])
    plsc.subcore_barrier()

    def body(j, _):
        pltpu.async_copy(hs_sh.at[sidx.at[j]], rows, sem).wait()
        pltpu.sync_copy(rows, acc_sh.at[didx.at[j]], add=True)
        return 0

    lax.fori_loop(0, CW, body, 0)
    plsc.subcore_barrier()
    pltpu.sync_copy(acc_sh.at[pl.ds(base, TR)],
                    out_hbm.at[pl.ds(c * NP + base, TR)])


def _tc1_body(x_ref, degc_ref, w1_ref, hs_ref):
    dis = lax.rsqrt(degc_ref[:, 0:1] + degc_ref[:, 1:2] + 1.0)
    h = jnp.dot(x_ref[...], w1_ref[...],
                preferred_element_type=jnp.float32,
                precision=lax.Precision.HIGHEST)
    hs_ref[...] = h * dis


def _tc2_body(pa_ref, pb_ref, hs_ref, degc_ref, b1_ref, w2_ref, hs2_ref):
    dis = lax.rsqrt(degc_ref[:, 0:1] + degc_ref[:, 1:2] + 1.0)
    agg = (pa_ref[...] + pb_ref[...] + hs_ref[...]) * dis
    y = jnp.maximum(agg + b1_ref[...], 0.0)
    hs2_ref[...] = jnp.dot(y, w2_ref[...],
                           preferred_element_type=jnp.float32,
                           precision=lax.Precision.HIGHEST) * dis


def _tc3_body(pa_ref, pb_ref, hs_ref, degc_ref, b2_ref,
              wf1_ref, bf1_ref, wf2_ref, bf2_ref, wf3_ref, bf3_ref, out_ref):
    dis = lax.rsqrt(degc_ref[:, 0:1] + degc_ref[:, 1:2] + 1.0)
    agg = (pa_ref[...] + pb_ref[...] + hs_ref[...]) * dis
    y = jnp.maximum(agg + b2_ref[...], 0.0)
    z = jnp.maximum(jnp.dot(y, wf1_ref[...],
                            preferred_element_type=jnp.float32,
                            precision=lax.Precision.HIGHEST) + bf1_ref[...], 0.0)
    z = jnp.maximum(jnp.dot(z, wf2_ref[...],
                            preferred_element_type=jnp.float32,
                            precision=lax.Precision.HIGHEST) + bf2_ref[...], 0.0)
    logits = jnp.dot(z, wf3_ref[...],
                     preferred_element_type=jnp.float32,
                     precision=lax.Precision.HIGHEST) + bf3_ref[...]
    mask = lax.broadcasted_iota(jnp.int32, (BLK, 128), 1) < NCLS
    neg = jnp.float32(-1e30)
    m = jnp.max(jnp.where(mask, logits, neg), axis=1, keepdims=True)
    ex = jnp.where(mask, jnp.exp(logits - m), 0.0)
    lse = jnp.log(jnp.sum(ex, axis=1, keepdims=True))
    out_ref[...] = logits - m - lse


def _row_spec(d):
    return pl.BlockSpec((BLK, d), lambda i: (i, 0))


def _full_spec(r, d):
    return pl.BlockSpec((r, d), lambda i: (0, 0))


def kernel(x, edge_index, W1, b1, W2, b2, Wf1, bf1, Wf2, bf2, Wf3, bf3):
    src = edge_index[0]
    dst = edge_index[1]
    pad = N + (jnp.arange(EPAD - E, dtype=jnp.int32) % (NP - N))
    src_p = jnp.concatenate([src, pad]).reshape(NW, CW, CB)
    dst_p = jnp.concatenate([dst, pad]).reshape(NW, CW, CB)
    x_p = jnp.pad(x, ((0, NP - N), (0, 0)))

    deg_flat = _get_sc_degree()(dst_p)
    degc = deg_flat.reshape(NC, NP).T

    hs1 = pl.pallas_call(
        _tc1_body,
        grid=(GRID,),
        in_specs=[_row_spec(D_IN), _row_spec(NC), _full_spec(D_IN, DH)],
        out_specs=_row_spec(DH),
        out_shape=jax.ShapeDtypeStruct((NP, DH), jnp.float32),
    )(x_p, degc, W1)

    part1 = _get_sc_aggregate()(hs1, src_p, dst_p)
    pa1, pb1 = part1[:NP], part1[NP:]

    b1r = b1.reshape(1, DH)
    hs2 = pl.pallas_call(
        _tc2_body,
        grid=(GRID,),
        in_specs=[_row_spec(DH), _row_spec(DH), _row_spec(DH), _row_spec(NC),
                  _full_spec(1, DH), _full_spec(DH, DH)],
        out_specs=_row_spec(DH),
        out_shape=jax.ShapeDtypeStruct((NP, DH), jnp.float32),
    )(pa1, pb1, hs1, degc, b1r, W2)

    part2 = _get_sc_aggregate()(hs2, src_p, dst_p)
    pa2, pb2 = part2[:NP], part2[NP:]

    wf3p = jnp.pad(Wf3, ((0, 0), (0, 128 - NCLS)))
    bf3p = jnp.pad(bf3, (0, 128 - NCLS)).reshape(1, 128)
    out = pl.pallas_call(
        _tc3_body,
        grid=(GRID,),
        in_specs=[_row_spec(DH), _row_spec(DH), _row_spec(DH), _row_spec(NC),
                  _full_spec(1, DH), _full_spec(DH, DH), _full_spec(1, DH),
                  _full_spec(DH, DH), _full_spec(1, DH),
                  _full_spec(DH, 128), _full_spec(1, 128)],
        out_specs=_row_spec(128),
        out_shape=jax.ShapeDtypeStruct((NP, 128), jnp.float32),
    )(pa2, pb2, hs2, degc, b2.reshape(1, DH), Wf1, bf1.reshape(1, DH),
      Wf2, bf2.reshape(1, DH), wf3p, bf3p)

    return out[:N, :NCLS]


# speedup vs baseline: 33.7035x; 1.2401x over previous
import functools

import jax
import jax.numpy as jnp
from jax import lax
from jax.experimental import pallas as pl
from jax.experimental.pallas import tpu as pltpu
from jax.experimental.pallas import tpu_sc as plsc

N = 10000
E = 320000
D_IN = 128
DH = 64
NCLS = 7

NC = 2
NS = 16
NW = NC * NS
CB = 128
CW = 80
EW = CW * CB
EPAD = NW * EW
NP = 10240
TR = NP // NS
BLK = 1024
GRID = NP // BLK

@functools.cache
def _get_sc_degree():
    mesh = plsc.VectorSubcoreMesh(core_axis_name="c", subcore_axis_name="s",
                                  num_cores=NC, num_subcores=NS)
    return pl.kernel(
        _sc_degree_body,
        out_type=jax.ShapeDtypeStruct((NC * NP,), jnp.float32),
        mesh=mesh,
        compiler_params=pltpu.CompilerParams(use_tc_tiling_on_sc=False),
        scratch_types=[
            pltpu.VMEM((CW, CB), jnp.int32),
            pltpu.VMEM((TR,), jnp.float32),
            pltpu.VMEM((CB,), jnp.float32),
            pltpu.VMEM_SHARED((NP,), jnp.float32),
        ],
    )


def _sc_degree_body(dst_hbm, out_hbm, didx, zbuf, ones, deg_sh):
    c = lax.axis_index("c")
    s = lax.axis_index("s")
    wid = c * NS + s
    base = s * TR

    def fill_z(i, _):
        zbuf[pl.ds(i * 16, 16)] = jnp.zeros((16,), jnp.float32)
        return 0

    lax.fori_loop(0, TR // 16, fill_z, 0)

    def fill_o(i, _):
        ones[pl.ds(i * 16, 16)] = jnp.ones((16,), jnp.float32)
        return 0

    lax.fori_loop(0, CB // 16, fill_o, 0)

    pltpu.sync_copy(dst_hbm.at[wid], didx)
    pltpu.sync_copy(zbuf, deg_sh.at[pl.ds(base, TR)])
    plsc.subcore_barrier()

    def body(j, _):
        pltpu.sync_copy(ones, deg_sh.at[didx.at[j]], add=True)
        return 0

    lax.fori_loop(0, CW, body, 0)
    plsc.subcore_barrier()
    pltpu.sync_copy(deg_sh.at[pl.ds(base, TR)],
                    out_hbm.at[pl.ds(c * NP + base, TR)])


@functools.cache
def _get_sc_aggregate():
    mesh = plsc.VectorSubcoreMesh(core_axis_name="c", subcore_axis_name="s",
                                  num_cores=NC, num_subcores=NS)
    return pl.kernel(
        _sc_aggregate_body,
        out_type=jax.ShapeDtypeStruct((NC * NP, DH), jnp.float32),
        mesh=mesh,
        compiler_params=pltpu.CompilerParams(use_tc_tiling_on_sc=False),
        scratch_types=[
            pltpu.VMEM((CW, CB), jnp.int32),
            pltpu.VMEM((CW, CB), jnp.int32),
            pltpu.VMEM((CB, DH), jnp.float32),
            pltpu.VMEM((CB, DH), jnp.float32),
            pltpu.VMEM_SHARED((NP, DH), jnp.float32),
            pltpu.VMEM_SHARED((NP, DH), jnp.float32),
            pltpu.SemaphoreType.DMA,
            pltpu.SemaphoreType.DMA,
            pltpu.SemaphoreType.DMA,
            pltpu.SemaphoreType.DMA,
        ],
    )


def _sc_aggregate_body(hs_hbm, src_hbm, dst_hbm, out_hbm,
                       sidx, didx, buf0, buf1, hs_sh, acc_sh,
                       gs0, gs1, ss0, ss1):
    c = lax.axis_index("c")
    s = lax.axis_index("s")
    wid = c * NS + s
    base = s * TR

    def fill_z(i, _):
        buf0[i // 4, pl.ds((i % 4) * 16, 16)] = jnp.zeros((16,), jnp.float32)
        return 0

    lax.fori_loop(0, CB * DH // 16, fill_z, 0)
    for t in range(TR // CB):
        pltpu.sync_copy(buf0, acc_sh.at[pl.ds(base + t * CB, CB)])

    pltpu.sync_copy(src_hbm.at[wid], sidx)
    pltpu.sync_copy(dst_hbm.at[wid], didx)
    pltpu.sync_copy(hs_hbm.at[pl.ds(base, TR)], hs_sh.at[pl.ds(base, TR)])
    plsc.subcore_barrier()

    pltpu.async_copy(hs_sh.at[sidx.at[0]], buf0, gs0)
    pltpu.async_copy(hs_sh.at[sidx.at[1]], buf1, gs1)

    def body(jj, _):
        j0 = 2 * jj
        j1 = j0 + 1
        pltpu.make_async_copy(hs_sh.at[sidx.at[j0]], buf0, gs0).wait()
        pltpu.async_copy(buf0, acc_sh.at[didx.at[j0]], ss0, add=True)
        pltpu.make_async_copy(hs_sh.at[sidx.at[j1]], buf1, gs1).wait()
        pltpu.async_copy(buf1, acc_sh.at[didx.at[j1]], ss1, add=True)

        @pl.when(jj < CW // 2 - 1)
        def _():
            pltpu.make_async_copy(buf0, acc_sh.at[didx.at[j0]], ss0).wait()
            pltpu.async_copy(hs_sh.at[sidx.at[j0 + 2]], buf0, gs0)
            pltpu.make_async_copy(buf1, acc_sh.at[didx.at[j1]], ss1).wait()
            pltpu.async_copy(hs_sh.at[sidx.at[j1 + 2]], buf1, gs1)

        return 0

    lax.fori_loop(0, CW // 2, body, 0)
    pltpu.make_async_copy(buf0, acc_sh.at[didx.at[CW - 2]], ss0).wait()
    pltpu.make_async_copy(buf1, acc_sh.at[didx.at[CW - 1]], ss1).wait()
    plsc.subcore_barrier()
    pltpu.sync_copy(acc_sh.at[pl.ds(base, TR)],
                    out_hbm.at[pl.ds(c * NP + base, TR)])


def _tc1_body(x_ref, degc_ref, w1_ref, hs_ref):
    dis = lax.rsqrt(degc_ref[:, 0:1] + degc_ref[:, 1:2] + 1.0)
    h = jnp.dot(x_ref[...], w1_ref[...],
                preferred_element_type=jnp.float32,
                precision=lax.Precision.HIGHEST)
    hs_ref[...] = h * dis


def _tc2_body(pa_ref, pb_ref, hs_ref, degc_ref, b1_ref, w2_ref, hs2_ref):
    dis = lax.rsqrt(degc_ref[:, 0:1] + degc_ref[:, 1:2] + 1.0)
    agg = (pa_ref[...] + pb_ref[...] + hs_ref[...]) * dis
    y = jnp.maximum(agg + b1_ref[...], 0.0)
    hs2_ref[...] = jnp.dot(y, w2_ref[...],
                           preferred_element_type=jnp.float32,
                           precision=lax.Precision.HIGHEST) * dis


def _tc3_body(pa_ref, pb_ref, hs_ref, degc_ref, b2_ref,
              wf1_ref, bf1_ref, wf2_ref, bf2_ref, wf3_ref, bf3_ref, out_ref):
    dis = lax.rsqrt(degc_ref[:, 0:1] + degc_ref[:, 1:2] + 1.0)
    agg = (pa_ref[...] + pb_ref[...] + hs_ref[...]) * dis
    y = jnp.maximum(agg + b2_ref[...], 0.0)
    z = jnp.maximum(jnp.dot(y, wf1_ref[...],
                            preferred_element_type=jnp.float32,
                            precision=lax.Precision.HIGHEST) + bf1_ref[...], 0.0)
    z = jnp.maximum(jnp.dot(z, wf2_ref[...],
                            preferred_element_type=jnp.float32,
                            precision=lax.Precision.HIGHEST) + bf2_ref[...], 0.0)
    logits = jnp.dot(z, wf3_ref[...],
                     preferred_element_type=jnp.float32,
                     precision=lax.Precision.HIGHEST) + bf3_ref[...]
    mask = lax.broadcasted_iota(jnp.int32, (BLK, 128), 1) < NCLS
    neg = jnp.float32(-1e30)
    m = jnp.max(jnp.where(mask, logits, neg), axis=1, keepdims=True)
    ex = jnp.where(mask, jnp.exp(logits - m), 0.0)
    lse = jnp.log(jnp.sum(ex, axis=1, keepdims=True))
    out_ref[...] = logits - m - lse


def _row_spec(d):
    return pl.BlockSpec((BLK, d), lambda i: (i, 0))


def _part_spec(core):
    off = core * (NP // BLK)
    return pl.BlockSpec((BLK, DH), lambda i: (off + i, 0))


def _full_spec(r, d):
    return pl.BlockSpec((r, d), lambda i: (0, 0))


def kernel(x, edge_index, W1, b1, W2, b2, Wf1, bf1, Wf2, bf2, Wf3, bf3):
    src = edge_index[0]
    dst = edge_index[1]
    pad = N + (jnp.arange(EPAD - E, dtype=jnp.int32) % (NP - N))
    src_p = jnp.concatenate([src, pad]).reshape(NW, CW, CB)
    dst_p = jnp.concatenate([dst, pad]).reshape(NW, CW, CB)
    x_p = jnp.pad(x, ((0, NP - N), (0, 0)))

    deg_flat = _get_sc_degree()(dst_p)
    degc = deg_flat.reshape(NC, NP).T

    hs1 = pl.pallas_call(
        _tc1_body,
        grid=(GRID,),
        in_specs=[_row_spec(D_IN), _row_spec(NC), _full_spec(D_IN, DH)],
        out_specs=_row_spec(DH),
        out_shape=jax.ShapeDtypeStruct((NP, DH), jnp.float32),
    )(x_p, degc, W1)

    part1 = _get_sc_aggregate()(hs1, src_p, dst_p)

    b1r = b1.reshape(1, DH)
    hs2 = pl.pallas_call(
        _tc2_body,
        grid=(GRID,),
        in_specs=[_part_spec(0), _part_spec(1), _row_spec(DH), _row_spec(NC),
                  _full_spec(1, DH), _full_spec(DH, DH)],
        out_specs=_row_spec(DH),
        out_shape=jax.ShapeDtypeStruct((NP, DH), jnp.float32),
    )(part1, part1, hs1, degc, b1r, W2)

    part2 = _get_sc_aggregate()(hs2, src_p, dst_p)

    wf3p = jnp.pad(Wf3, ((0, 0), (0, 128 - NCLS)))
    bf3p = jnp.pad(bf3, (0, 128 - NCLS)).reshape(1, 128)
    out = pl.pallas_call(
        _tc3_body,
        grid=(GRID,),
        in_specs=[_part_spec(0), _part_spec(1), _row_spec(DH), _row_spec(NC),
                  _full_spec(1, DH), _full_spec(DH, DH), _full_spec(1, DH),
                  _full_spec(DH, DH), _full_spec(1, DH),
                  _full_spec(DH, 128), _full_spec(1, 128)],
        out_specs=_row_spec(128),
        out_shape=jax.ShapeDtypeStruct((NP, 128), jnp.float32),
    )(part2, part2, hs2, degc, b2.reshape(1, DH), Wf1, bf1.reshape(1, DH),
      Wf2, bf2.reshape(1, DH), wf3p, bf3p)

    return out[:N, :NCLS]


# speedup vs baseline: 36.7056x; 1.0891x over previous
import functools

import jax
import jax.numpy as jnp
from jax import lax
from jax.experimental import pallas as pl
from jax.experimental.pallas import tpu as pltpu
from jax.experimental.pallas import tpu_sc as plsc

N = 10000
E = 320000
D_IN = 128
DH = 64
NCLS = 7

NC = 2
NS = 16
NW = NC * NS
CB = 125
CW = 80
TR = N // NS
NPD = 10240
TRD = NPD // NS
BLK = 1000
GRID = N // BLK

_params = pltpu.CompilerParams(use_tc_tiling_on_sc=False)


@functools.cache
def _get_sc_degree():
    mesh = plsc.VectorSubcoreMesh(core_axis_name="c", subcore_axis_name="s",
                                  num_cores=NC, num_subcores=NS)
    return pl.kernel(
        _sc_degree_body,
        out_type=jax.ShapeDtypeStruct((NC * NPD,), jnp.float32),
        mesh=mesh,
        compiler_params=_params,
        scratch_types=[
            pltpu.VMEM((CW, CB), jnp.int32),
            pltpu.VMEM((TRD,), jnp.float32),
            pltpu.VMEM((128,), jnp.float32),
            pltpu.VMEM_SHARED((NPD,), jnp.float32),
        ],
    )


def _sc_degree_body(dst_hbm, out_hbm, didx, zbuf, ones, deg_sh):
    c = lax.axis_index("c")
    s = lax.axis_index("s")
    wid = c * NS + s
    base = s * TRD

    def fill_z(i, _):
        zbuf[pl.ds(i * 16, 16)] = jnp.zeros((16,), jnp.float32)
        return 0

    lax.fori_loop(0, TRD // 16, fill_z, 0)

    def fill_o(i, _):
        ones[pl.ds(i * 16, 16)] = jnp.ones((16,), jnp.float32)
        return 0

    lax.fori_loop(0, 8, fill_o, 0)

    pltpu.sync_copy(dst_hbm.at[wid], didx)
    pltpu.sync_copy(zbuf, deg_sh.at[pl.ds(base, TRD)])
    plsc.subcore_barrier()

    def body(j, _):
        pltpu.sync_copy(ones.at[pl.ds(0, CB)], deg_sh.at[didx.at[j]], add=True)
        return 0

    lax.fori_loop(0, CW, body, 0)
    plsc.subcore_barrier()
    pltpu.sync_copy(deg_sh.at[pl.ds(base, TRD)],
                    out_hbm.at[pl.ds(c * NPD + base, TRD)])


@functools.cache
def _get_sc_aggregate():
    mesh = plsc.VectorSubcoreMesh(core_axis_name="c", subcore_axis_name="s",
                                  num_cores=NC, num_subcores=NS)
    return pl.kernel(
        _sc_aggregate_body,
        out_type=jax.ShapeDtypeStruct((NC * N, DH), jnp.float32),
        mesh=mesh,
        compiler_params=_params,
        scratch_types=[
            pltpu.VMEM((CW, CB), jnp.int32),
            pltpu.VMEM((CW, CB), jnp.int32),
            pltpu.VMEM((CB, DH), jnp.float32),
            pltpu.VMEM((CB, DH), jnp.float32),
            pltpu.VMEM((CB, DH), jnp.float32),
            pltpu.VMEM((CB, DH), jnp.float32),
            pltpu.VMEM_SHARED((NPD, DH), jnp.float32),
            pltpu.SemaphoreType.DMA,
            pltpu.SemaphoreType.DMA,
            pltpu.SemaphoreType.DMA,
            pltpu.SemaphoreType.DMA,
            pltpu.SemaphoreType.DMA,
            pltpu.SemaphoreType.DMA,
            pltpu.SemaphoreType.DMA,
            pltpu.SemaphoreType.DMA,
        ],
    )


_NBUF = 4


def _sc_aggregate_body(hs_hbm, src_hbm, dst_hbm, out_hbm,
                       sidx, didx, b0, b1, b2, b3, acc_sh,
                       g0, g1, g2, g3, s0, s1, s2, s3):
    c = lax.axis_index("c")
    s = lax.axis_index("s")
    wid = c * NS + s
    base = s * TR
    bufs = (b0, b1, b2, b3)
    gsems = (g0, g1, g2, g3)
    ssems = (s0, s1, s2, s3)

    def fill_z(i, _):
        b0[i // 4, pl.ds((i % 4) * 16, 16)] = jnp.zeros((16,), jnp.float32)
        return 0

    lax.fori_loop(0, CB * DH // 16, fill_z, 0)
    for t in range(TR // CB):
        pltpu.sync_copy(b0, acc_sh.at[pl.ds(base + t * CB, CB)])

    pltpu.sync_copy(src_hbm.at[wid], sidx)
    pltpu.sync_copy(dst_hbm.at[wid], didx)
    plsc.subcore_barrier()

    for k in range(_NBUF):
        pltpu.async_copy(hs_hbm.at[sidx.at[k]], bufs[k], gsems[k])

    def body(jj, _):
        jb = jj * _NBUF
        for k in range(_NBUF):
            j = jb + k
            pltpu.make_async_copy(hs_hbm.at[sidx.at[j]], bufs[k],
                                  gsems[k]).wait()
            pltpu.async_copy(bufs[k], acc_sh.at[didx.at[j]], ssems[k],
                             add=True)

        @pl.when(jj < CW // _NBUF - 1)
        def _():
            for k in range(_NBUF):
                j = jb + k
                pltpu.make_async_copy(bufs[k], acc_sh.at[didx.at[j]],
                                      ssems[k]).wait()
                pltpu.async_copy(hs_hbm.at[sidx.at[j + _NBUF]], bufs[k],
                                 gsems[k])

        return 0

    lax.fori_loop(0, CW // _NBUF, body, 0)
    for k in range(_NBUF):
        pltpu.make_async_copy(bufs[k], acc_sh.at[didx.at[CW - _NBUF + k]],
                              ssems[k]).wait()
    plsc.subcore_barrier()
    pltpu.sync_copy(acc_sh.at[pl.ds(base, TR)],
                    out_hbm.at[pl.ds(c * N + base, TR)])


def _tc1_body(x_ref, degc_ref, w1_ref, hs_ref):
    dis = lax.rsqrt(degc_ref[:, 0:1] + degc_ref[:, 1:2] + 1.0)
    h = jnp.dot(x_ref[...], w1_ref[...],
                preferred_element_type=jnp.float32,
                precision=lax.Precision.HIGHEST)
    hs_ref[...] = h * dis


def _tc2_body(pa_ref, pb_ref, hs_ref, degc_ref, b1_ref, w2_ref, hs2_ref):
    dis = lax.rsqrt(degc_ref[:, 0:1] + degc_ref[:, 1:2] + 1.0)
    agg = (pa_ref[...] + pb_ref[...] + hs_ref[...]) * dis
    y = jnp.maximum(agg + b1_ref[...], 0.0)
    hs2_ref[...] = jnp.dot(y, w2_ref[...],
                           preferred_element_type=jnp.float32,
                           precision=lax.Precision.HIGHEST) * dis


def _tc3_body(pa_ref, pb_ref, hs_ref, degc_ref, b2_ref,
              wf1_ref, bf1_ref, wf2_ref, bf2_ref, wf3_ref, bf3_ref, out_ref):
    dis = lax.rsqrt(degc_ref[:, 0:1] + degc_ref[:, 1:2] + 1.0)
    agg = (pa_ref[...] + pb_ref[...] + hs_ref[...]) * dis
    y = jnp.maximum(agg + b2_ref[...], 0.0)
    z = jnp.maximum(jnp.dot(y, wf1_ref[...],
                            preferred_element_type=jnp.float32,
                            precision=lax.Precision.HIGHEST) + bf1_ref[...], 0.0)
    z = jnp.maximum(jnp.dot(z, wf2_ref[...],
                            preferred_element_type=jnp.float32,
                            precision=lax.Precision.HIGHEST) + bf2_ref[...], 0.0)
    logits = jnp.dot(z, wf3_ref[...],
                     preferred_element_type=jnp.float32,
                     precision=lax.Precision.HIGHEST) + bf3_ref[...]
    mask = lax.broadcasted_iota(jnp.int32, (BLK, 128), 1) < NCLS
    neg = jnp.float32(-1e30)
    m = jnp.max(jnp.where(mask, logits, neg), axis=1, keepdims=True)
    ex = jnp.where(mask, jnp.exp(logits - m), 0.0)
    lse = jnp.log(jnp.sum(ex, axis=1, keepdims=True))
    out_ref[...] = logits - m - lse


def _row_spec(d):
    return pl.BlockSpec((BLK, d), lambda i: (i, 0))


def _part_spec(core):
    off = core * (N // BLK)
    return pl.BlockSpec((BLK, DH), lambda i: (off + i, 0))


def _full_spec(r, d):
    return pl.BlockSpec((r, d), lambda i: (0, 0))


def kernel(x, edge_index, W1, b1, W2, b2, Wf1, bf1, Wf2, bf2, Wf3, bf3):
    src_p = edge_index[0].reshape(NW, CW, CB)
    dst_p = edge_index[1].reshape(NW, CW, CB)

    deg_flat = _get_sc_degree()(dst_p)
    degc = deg_flat.reshape(NC, NPD).T

    hs1 = pl.pallas_call(
        _tc1_body,
        grid=(GRID,),
        in_specs=[_row_spec(D_IN), _row_spec(NC), _full_spec(D_IN, DH)],
        out_specs=_row_spec(DH),
        out_shape=jax.ShapeDtypeStruct((N, DH), jnp.float32),
    )(x, degc, W1)

    part1 = _get_sc_aggregate()(hs1, src_p, dst_p)

    b1r = b1.reshape(1, DH)
    hs2 = pl.pallas_call(
        _tc2_body,
        grid=(GRID,),
        in_specs=[_part_spec(0), _part_spec(1), _row_spec(DH), _row_spec(NC),
                  _full_spec(1, DH), _full_spec(DH, DH)],
        out_specs=_row_spec(DH),
        out_shape=jax.ShapeDtypeStruct((N, DH), jnp.float32),
    )(part1, part1, hs1, degc, b1r, W2)

    part2 = _get_sc_aggregate()(hs2, src_p, dst_p)

    wf3p = jnp.pad(Wf3, ((0, 0), (0, 128 - NCLS)))
    bf3p = jnp.pad(bf3, (0, 128 - NCLS)).reshape(1, 128)
    out = pl.pallas_call(
        _tc3_body,
        grid=(GRID,),
        in_specs=[_part_spec(0), _part_spec(1), _row_spec(DH), _row_spec(NC),
                  _full_spec(1, DH), _full_spec(DH, DH), _full_spec(1, DH),
                  _full_spec(DH, DH), _full_spec(1, DH),
                  _full_spec(DH, 128), _full_spec(1, 128)],
        out_specs=_row_spec(128),
        out_shape=jax.ShapeDtypeStruct((N, 128), jnp.float32),
    )(part2, part2, hs2, degc, b2.reshape(1, DH), Wf1, bf1.reshape(1, DH),
      Wf2, bf2.reshape(1, DH), wf3p, bf3p)

    return out[:, :NCLS]


# speedup vs baseline: 44.5509x; 1.2137x over previous
import functools

import jax
import jax.numpy as jnp
from jax import lax
from jax.experimental import pallas as pl
from jax.experimental.pallas import tpu as pltpu
from jax.experimental.pallas import tpu_sc as plsc

N = 10000
E = 320000
D_IN = 128
DH = 64
NCLS = 7

NC = 2
NS = 16
NW = NC * NS
CB = 125
CW = 80
TR = N // NS
NPD = 10240
TRD = NPD // NS
BLK = 2000
GRID = N // BLK

_params = pltpu.CompilerParams(use_tc_tiling_on_sc=False)


@functools.cache
def _get_sc_degree():
    mesh = plsc.VectorSubcoreMesh(core_axis_name="c", subcore_axis_name="s",
                                  num_cores=NC, num_subcores=NS)
    return pl.kernel(
        _sc_degree_body,
        out_type=jax.ShapeDtypeStruct((NC * NPD,), jnp.float32),
        mesh=mesh,
        compiler_params=_params,
        scratch_types=[
            pltpu.VMEM((CW, CB), jnp.int32),
            pltpu.VMEM((TRD,), jnp.float32),
            pltpu.VMEM((128,), jnp.float32),
            pltpu.VMEM_SHARED((NPD,), jnp.float32),
        ],
    )


def _sc_degree_body(dst_hbm, out_hbm, didx, zbuf, ones, deg_sh):
    c = lax.axis_index("c")
    s = lax.axis_index("s")
    wid = c * NS + s
    base = s * TRD

    def fill_z(i, _):
        zbuf[pl.ds(i * 16, 16)] = jnp.zeros((16,), jnp.float32)
        return 0

    lax.fori_loop(0, TRD // 16, fill_z, 0)

    def fill_o(i, _):
        ones[pl.ds(i * 16, 16)] = jnp.ones((16,), jnp.float32)
        return 0

    lax.fori_loop(0, 8, fill_o, 0)

    pltpu.sync_copy(dst_hbm.at[wid], didx)
    pltpu.sync_copy(zbuf, deg_sh.at[pl.ds(base, TRD)])
    plsc.subcore_barrier()

    def body(j, _):
        pltpu.sync_copy(ones.at[pl.ds(0, CB)], deg_sh.at[didx.at[j]], add=True)
        return 0

    lax.fori_loop(0, CW, body, 0)
    plsc.subcore_barrier()
    pltpu.sync_copy(deg_sh.at[pl.ds(base, TRD)],
                    out_hbm.at[pl.ds(c * NPD + base, TRD)])


@functools.cache
def _get_sc_aggregate():
    mesh = plsc.VectorSubcoreMesh(core_axis_name="c", subcore_axis_name="s",
                                  num_cores=NC, num_subcores=NS)
    return pl.kernel(
        _sc_aggregate_body,
        out_type=jax.ShapeDtypeStruct((NC * N, DH), jnp.float32),
        mesh=mesh,
        compiler_params=_params,
        scratch_types=[
            pltpu.VMEM((CW, CB), jnp.int32),
            pltpu.VMEM((CW, CB), jnp.int32),
            pltpu.VMEM((CB, DH), jnp.float32),
            pltpu.VMEM((CB, DH), jnp.float32),
            pltpu.VMEM((CB, DH), jnp.float32),
            pltpu.VMEM((CB, DH), jnp.float32),
            pltpu.VMEM((CB, DH), jnp.float32),
            pltpu.VMEM((CB, DH), jnp.float32),
            pltpu.VMEM((CB, DH), jnp.float32),
            pltpu.VMEM((CB, DH), jnp.float32),
            pltpu.VMEM_SHARED((NPD, DH), jnp.float32),
            pltpu.SemaphoreType.DMA,
            pltpu.SemaphoreType.DMA,
            pltpu.SemaphoreType.DMA,
            pltpu.SemaphoreType.DMA,
            pltpu.SemaphoreType.DMA,
            pltpu.SemaphoreType.DMA,
            pltpu.SemaphoreType.DMA,
            pltpu.SemaphoreType.DMA,
            pltpu.SemaphoreType.DMA,
            pltpu.SemaphoreType.DMA,
            pltpu.SemaphoreType.DMA,
            pltpu.SemaphoreType.DMA,
            pltpu.SemaphoreType.DMA,
            pltpu.SemaphoreType.DMA,
            pltpu.SemaphoreType.DMA,
            pltpu.SemaphoreType.DMA,
        ],
    )


_NBUF = 8


def _sc_aggregate_body(hs_hbm, src_hbm, dst_hbm, out_hbm,
                       sidx, didx, b0, b1, b2, b3, b4, b5, b6, b7, acc_sh,
                       g0, g1, g2, g3, g4, g5, g6, g7,
                       s0, s1, s2, s3, s4, s5, s6, s7):
    c = lax.axis_index("c")
    s = lax.axis_index("s")
    wid = c * NS + s
    base = s * TR
    bufs = (b0, b1, b2, b3, b4, b5, b6, b7)
    gsems = (g0, g1, g2, g3, g4, g5, g6, g7)
    ssems = (s0, s1, s2, s3, s4, s5, s6, s7)

    def fill_z(i, _):
        b0[i // 4, pl.ds((i % 4) * 16, 16)] = jnp.zeros((16,), jnp.float32)
        return 0

    lax.fori_loop(0, CB * DH // 16, fill_z, 0)
    for t in range(TR // CB):
        pltpu.sync_copy(b0, acc_sh.at[pl.ds(base + t * CB, CB)])

    pltpu.sync_copy(src_hbm.at[wid], sidx)
    pltpu.sync_copy(dst_hbm.at[wid], didx)
    plsc.subcore_barrier()

    for k in range(_NBUF):
        pltpu.async_copy(hs_hbm.at[sidx.at[k]], bufs[k], gsems[k])

    def body(jj, _):
        jb = jj * _NBUF
        for k in range(_NBUF):
            j = jb + k
            pltpu.make_async_copy(hs_hbm.at[sidx.at[j]], bufs[k],
                                  gsems[k]).wait()
            pltpu.async_copy(bufs[k], acc_sh.at[didx.at[j]], ssems[k],
                             add=True)

        @pl.when(jj < CW // _NBUF - 1)
        def _():
            for k in range(_NBUF):
                j = jb + k
                pltpu.make_async_copy(bufs[k], acc_sh.at[didx.at[j]],
                                      ssems[k]).wait()
                pltpu.async_copy(hs_hbm.at[sidx.at[j + _NBUF]], bufs[k],
                                 gsems[k])

        return 0

    lax.fori_loop(0, CW // _NBUF, body, 0)
    for k in range(_NBUF):
        pltpu.make_async_copy(bufs[k], acc_sh.at[didx.at[CW - _NBUF + k]],
                              ssems[k]).wait()
    plsc.subcore_barrier()
    pltpu.sync_copy(acc_sh.at[pl.ds(base, TR)],
                    out_hbm.at[pl.ds(c * N + base, TR)])


def _tc1_body(x_ref, degc_ref, w1_ref, hs_ref):
    dis = lax.rsqrt(degc_ref[:, 0:1] + degc_ref[:, 1:2] + 1.0)
    h = jnp.dot(x_ref[...], w1_ref[...],
                preferred_element_type=jnp.float32)
    hs_ref[...] = h * dis


def _tc2_body(pa_ref, pb_ref, hs_ref, degc_ref, b1_ref, w2_ref, hs2_ref):
    dis = lax.rsqrt(degc_ref[:, 0:1] + degc_ref[:, 1:2] + 1.0)
    agg = (pa_ref[...] + pb_ref[...] + hs_ref[...]) * dis
    y = jnp.maximum(agg + b1_ref[...], 0.0)
    hs2_ref[...] = jnp.dot(y, w2_ref[...],
                           preferred_element_type=jnp.float32) * dis


def _tc3_body(pa_ref, pb_ref, hs_ref, degc_ref, b2_ref,
              wf1_ref, bf1_ref, wf2_ref, bf2_ref, wf3_ref, bf3_ref, out_ref):
    dis = lax.rsqrt(degc_ref[:, 0:1] + degc_ref[:, 1:2] + 1.0)
    agg = (pa_ref[...] + pb_ref[...] + hs_ref[...]) * dis
    y = jnp.maximum(agg + b2_ref[...], 0.0)
    z = jnp.maximum(jnp.dot(y, wf1_ref[...],
                            preferred_element_type=jnp.float32) + bf1_ref[...], 0.0)
    z = jnp.maximum(jnp.dot(z, wf2_ref[...],
                            preferred_element_type=jnp.float32) + bf2_ref[...], 0.0)
    logits = jnp.dot(z, wf3_ref[...],
                     preferred_element_type=jnp.float32) + bf3_ref[...]
    mask = lax.broadcasted_iota(jnp.int32, (BLK, 128), 1) < NCLS
    neg = jnp.float32(-1e30)
    m = jnp.max(jnp.where(mask, logits, neg), axis=1, keepdims=True)
    ex = jnp.where(mask, jnp.exp(logits - m), 0.0)
    lse = jnp.log(jnp.sum(ex, axis=1, keepdims=True))
    out_ref[...] = logits - m - lse


def _row_spec(d):
    return pl.BlockSpec((BLK, d), lambda i: (i, 0))


def _part_spec(core):
    off = core * (N // BLK)
    return pl.BlockSpec((BLK, DH), lambda i: (off + i, 0))


def _full_spec(r, d):
    return pl.BlockSpec((r, d), lambda i: (0, 0))


def kernel(x, edge_index, W1, b1, W2, b2, Wf1, bf1, Wf2, bf2, Wf3, bf3):
    src_p = edge_index[0].reshape(NW, CW, CB)
    dst_p = edge_index[1].reshape(NW, CW, CB)

    deg_flat = _get_sc_degree()(dst_p)
    degc = deg_flat.reshape(NC, NPD).T

    hs1 = pl.pallas_call(
        _tc1_body,
        grid=(GRID,),
        in_specs=[_row_spec(D_IN), _row_spec(NC), _full_spec(D_IN, DH)],
        out_specs=_row_spec(DH),
        out_shape=jax.ShapeDtypeStruct((N, DH), jnp.float32),
    )(x, degc, W1)

    part1 = _get_sc_aggregate()(hs1, src_p, dst_p)

    b1r = b1.reshape(1, DH)
    hs2 = pl.pallas_call(
        _tc2_body,
        grid=(GRID,),
        in_specs=[_part_spec(0), _part_spec(1), _row_spec(DH), _row_spec(NC),
                  _full_spec(1, DH), _full_spec(DH, DH)],
        out_specs=_row_spec(DH),
        out_shape=jax.ShapeDtypeStruct((N, DH), jnp.float32),
    )(part1, part1, hs1, degc, b1r, W2)

    part2 = _get_sc_aggregate()(hs2, src_p, dst_p)

    wf3p = jnp.pad(Wf3, ((0, 0), (0, 128 - NCLS)))
    bf3p = jnp.pad(bf3, (0, 128 - NCLS)).reshape(1, 128)
    out = pl.pallas_call(
        _tc3_body,
        grid=(GRID,),
        in_specs=[_part_spec(0), _part_spec(1), _row_spec(DH), _row_spec(NC),
                  _full_spec(1, DH), _full_spec(DH, DH), _full_spec(1, DH),
                  _full_spec(DH, DH), _full_spec(1, DH),
                  _full_spec(DH, 128), _full_spec(1, 128)],
        out_specs=_row_spec(128),
        out_shape=jax.ShapeDtypeStruct((N, 128), jnp.float32),
    )(part2, part2, hs2, degc, b2.reshape(1, DH), Wf1, bf1.reshape(1, DH),
      Wf2, bf2.reshape(1, DH), wf3p, bf3p)

    return out[:, :NCLS]
